# Initial kernel scaffold; baseline (speedup 1.0000x reference)
import functools

import jax
import jax.numpy as jnp
from jax import lax
from jax.experimental import pallas as pl
from jax.experimental.pallas import tpu as pltpu
from jax.experimental.pallas import tpu_sc as plsc

_N = 10000
_E = 320000
_D = 128
_H = 64
_G = 64
_C = 10

_NSC = 2
_NTILE = 16
_NW = _NSC * _NTILE
_EPT = _E // _NW
_K = 100
_NCH = _EPT // _K
_RPT = _N // _NTILE

_NB = 10
_BN = _N // _NB


@functools.partial(
    pl.kernel,
    out_type=jax.ShapeDtypeStruct((_NSC, _N, _H), jnp.float32),
    mesh=plsc.VectorSubcoreMesh(core_axis_name="c", subcore_axis_name="s"),
    scratch_types=[
        pltpu.VMEM((_NCH, _K), jnp.int32),
        pltpu.VMEM((_NCH, _K), jnp.int32),
        pltpu.VMEM((_K, _H), jnp.float32),
        pltpu.VMEM_SHARED((_N, _H), jnp.float32),
        pltpu.SemaphoreType.DMA,
    ],
)
def _sc_agg(p_hbm, src_hbm, dst_hbm, zero_hbm, out_hbm, srcv, dstv, rows, acc, sem):
    c = lax.axis_index("c")
    s = lax.axis_index("s")
    wid = c * _NTILE + s
    pltpu.sync_copy(zero_hbm.at[pl.ds(s * _RPT, _RPT)], acc.at[pl.ds(s * _RPT, _RPT)])
    pltpu.sync_copy(src_hbm.at[wid], srcv)
    pltpu.sync_copy(dst_hbm.at[wid], dstv)
    plsc.subcore_barrier()

    def chunk(j, carry):
        pltpu.async_copy(p_hbm.at[srcv.at[j]], rows, sem).wait()
        pltpu.sync_copy(rows, acc.at[dstv.at[j]], add=True)
        return carry

    lax.fori_loop(0, _NCH, chunk, 0)
    plsc.subcore_barrier()
    pltpu.sync_copy(acc.at[pl.ds(s * _RPT, _RPT)],
                    out_hbm.at[c, pl.ds(s * _RPT, _RPT)])


def _proj_body(x_ref, w_ref, o_ref):
    o_ref[...] = jnp.dot(x_ref[...], w_ref[...], preferred_element_type=jnp.float32)


def _proj(x, w):
    d = x.shape[1]
    return pl.pallas_call(
        _proj_body,
        grid=(_NB,),
        in_specs=[
            pl.BlockSpec((_BN, d), lambda i: (i, 0)),
            pl.BlockSpec((d, _H), lambda i: (0, 0)),
        ],
        out_specs=pl.BlockSpec((_BN, _H), lambda i: (i, 0)),
        out_shape=jax.ShapeDtypeStruct((_N, _H), jnp.float32),
    )(x, w)


def _mlp_body(has_next, p_ref, agg_ref, b1_ref, w2_ref, b2_ref, w1n_ref, h_ref,
              pn_ref=None):
    m = 2.0 * p_ref[...] + agg_ref[0] + agg_ref[1] + b1_ref[...]
    m = jnp.maximum(m, 0.0)
    h = jnp.dot(m, w2_ref[...], preferred_element_type=jnp.float32) + b2_ref[...]
    h = jnp.maximum(h, 0.0)
    h_ref[...] = h
    if has_next:
        pn_ref[...] = jnp.dot(h, w1n_ref[...], preferred_element_type=jnp.float32)


def _mlp(p, agg, b1, w2, b2, w1n, has_next):
    in_specs = [
        pl.BlockSpec((_BN, _H), lambda i: (i, 0)),
        pl.BlockSpec((_NSC, _BN, _H), lambda i: (0, i, 0)),
        pl.BlockSpec((1, _H), lambda i: (0, 0)),
        pl.BlockSpec((_H, _H), lambda i: (0, 0)),
        pl.BlockSpec((1, _H), lambda i: (0, 0)),
        pl.BlockSpec((_H, _H), lambda i: (0, 0)),
    ]
    if has_next:
        out_specs = [pl.BlockSpec((_BN, _H), lambda i: (i, 0)),
                     pl.BlockSpec((_BN, _H), lambda i: (i, 0))]
        out_shape = [jax.ShapeDtypeStruct((_N, _H), jnp.float32),
                     jax.ShapeDtypeStruct((_N, _H), jnp.float32)]
    else:
        out_specs = pl.BlockSpec((_BN, _H), lambda i: (i, 0))
        out_shape = jax.ShapeDtypeStruct((_N, _H), jnp.float32)
    return pl.pallas_call(
        functools.partial(_mlp_body, has_next),
        grid=(_NB,),
        in_specs=in_specs,
        out_specs=out_specs,
        out_shape=out_shape,
    )(p, agg, b1, w2, b2, w1n)


def _pool_head_body(h1_ref, h2_ref, h3_ref, bat_ref, fc1w_ref, fc1b_ref,
                    fc2w_ref, fc2b_ref, out_ref, s_acc, m_acc, c_acc):
    i = pl.program_id(0)

    @pl.when(i == 0)
    def _init():
        s_acc[...] = jnp.zeros_like(s_acc)
        m_acc[...] = jnp.full_like(m_acc, -jnp.inf)
        c_acc[...] = jnp.zeros_like(c_acc)

    X = jnp.concatenate([h1_ref[...], h2_ref[...], h3_ref[...]], axis=1)
    bb = bat_ref[0, 0, :]
    onehot = (bb[:, None] == lax.broadcasted_iota(jnp.int32, (_BN, _G), 1)
              ).astype(jnp.float32)
    s_acc[...] += lax.dot_general(onehot, X, (((0,), (0,)), ((), ())),
                                  preferred_element_type=jnp.float32)
    ones = jnp.ones((_BN, 8), jnp.float32)
    c_acc[...] += lax.dot_general(onehot, ones, (((0,), (0,)), ((), ())),
                                  preferred_element_type=jnp.float32)

    def g_body(g, carry):
        mask = (bb == g)[:, None]
        mg = jnp.max(jnp.where(mask, X, -jnp.inf), axis=0, keepdims=True)
        m_acc[pl.ds(g, 1), :] = jnp.maximum(m_acc[pl.ds(g, 1), :], mg)
        return carry

    lax.fori_loop(0, _G, g_body, 0)

    @pl.when(i == _NB - 1)
    def _final():
        cnt = c_acc[:, 0:1]
        s = s_acc[...]
        mean = s / jnp.maximum(cnt, 1.0)
        mx = jnp.where(cnt > 0.0, m_acc[...], 0.0)
        w = fc1w_ref[...]
        z = (jnp.dot(mean, w[0:192], preferred_element_type=jnp.float32)
             + jnp.dot(mx, w[192:384], preferred_element_type=jnp.float32)
             + jnp.dot(s, w[384:576], preferred_element_type=jnp.float32)
             + fc1b_ref[...])
        z = jnp.maximum(z, 0.0)
        o = jnp.dot(z, fc2w_ref[...], preferred_element_type=jnp.float32) \
            + fc2b_ref[...]
        out_ref[...] = 1.0 / (1.0 + jnp.exp(-o))


def _pool_head(h1, h2, h3, bat3, fc1_w, fc1_b, fc2_w, fc2_b):
    return pl.pallas_call(
        _pool_head_body,
        grid=(_NB,),
        in_specs=[
            pl.BlockSpec((_BN, _H), lambda i: (i, 0)),
            pl.BlockSpec((_BN, _H), lambda i: (i, 0)),
            pl.BlockSpec((_BN, _H), lambda i: (i, 0)),
            pl.BlockSpec((1, 1, _BN), lambda i: (i, 0, 0)),
            pl.BlockSpec((9 * _H, _H), lambda i: (0, 0)),
            pl.BlockSpec((1, _H), lambda i: (0, 0)),
            pl.BlockSpec((_H, _C), lambda i: (0, 0)),
            pl.BlockSpec((1, _C), lambda i: (0, 0)),
        ],
        out_specs=pl.BlockSpec((_G, _C), lambda i: (0, 0)),
        out_shape=jax.ShapeDtypeStruct((_G, _C), jnp.float32),
        scratch_shapes=[
            pltpu.VMEM((_G, 3 * _H), jnp.float32),
            pltpu.VMEM((_G, 3 * _H), jnp.float32),
            pltpu.VMEM((_G, 8), jnp.float32),
        ],
    )(h1, h2, h3, bat3, fc1_w, fc1_b, fc2_w, fc2_b)


def kernel(x, edge_index, batch,
           c0_w1, c0_b1, c0_w2, c0_b2,
           c1_w1, c1_b1, c1_w2, c1_b2,
           c2_w1, c2_b1, c2_w2, c2_b2,
           fc1_w, fc1_b, fc2_w, fc2_b):
    src = edge_index[0].astype(jnp.int32).reshape(_NW, _NCH, _K)
    dst = edge_index[1].astype(jnp.int32).reshape(_NW, _NCH, _K)
    zeros = jnp.zeros((_N, _H), jnp.float32)
    bat3 = batch.astype(jnp.int32).reshape(_NB, 1, _BN)

    params = [(c0_b1, c0_w2, c0_b2), (c1_b1, c1_w2, c1_b2), (c2_b1, c2_w2, c2_b2)]
    next_w1 = [c1_w1, c2_w1, None]

    p = _proj(x, c0_w1)
    hs = []
    for l in range(3):
        agg = _sc_agg(p, src, dst, zeros)
        b1, w2, b2 = params[l]
        has_next = next_w1[l] is not None
        w1n = next_w1[l] if has_next else w2
        res = _mlp(p, agg, b1.reshape(1, _H), w2, b2.reshape(1, _H), w1n,
                   has_next)
        if has_next:
            h, p = res
        else:
            h = res
        hs.append(h)

    return _pool_head(hs[0], hs[1], hs[2], bat3, fc1_w,
                      fc1_b.reshape(1, _H), fc2_w, fc2_b.reshape(1, _C))


# speedup vs baseline: 6.8718x; 6.8718x over previous
import functools

import jax
import jax.numpy as jnp
from jax import lax
from jax.experimental import pallas as pl
from jax.experimental.pallas import tpu as pltpu
from jax.experimental.pallas import tpu_sc as plsc

_N = 10000
_E = 320000
_D = 128
_H = 64
_G = 64
_C = 10

_NSC = 2
_NTILE = 16
_NW = _NSC * _NTILE
_EPT = _E // _NW
_K = 100
_NCH = _EPT // _K
_NP = 10240
_RPT = _NP // _NTILE

_NB = 10
_BN = _N // _NB


@functools.partial(
    pl.kernel,
    out_type=jax.ShapeDtypeStruct((_NSC, _NP, _H), jnp.float32),
    mesh=plsc.VectorSubcoreMesh(core_axis_name="c", subcore_axis_name="s"),
    scratch_types=[
        pltpu.VMEM((_NCH, _K), jnp.int32),
        pltpu.VMEM((_NCH, _K), jnp.int32),
        pltpu.VMEM((_K, _H), jnp.float32),
        pltpu.VMEM_SHARED((_NP, _H), jnp.float32),
        pltpu.SemaphoreType.DMA,
    ],
    compiler_params=pltpu.CompilerParams(use_tc_tiling_on_sc=False),
)
def _sc_agg(p_hbm, src_hbm, dst_hbm, zero_hbm, out_hbm, srcv, dstv, rows, acc, sem):
    c = lax.axis_index("c")
    s = lax.axis_index("s")
    wid = c * _NTILE + s
    pltpu.sync_copy(zero_hbm.at[pl.ds(s * _RPT, _RPT)], acc.at[pl.ds(s * _RPT, _RPT)])
    pltpu.sync_copy(src_hbm.at[wid], srcv)
    pltpu.sync_copy(dst_hbm.at[wid], dstv)
    plsc.subcore_barrier()

    def chunk(j, carry):
        pltpu.async_copy(p_hbm.at[srcv.at[j]], rows, sem).wait()
        pltpu.sync_copy(rows, acc.at[dstv.at[j]], add=True)
        return carry

    lax.fori_loop(0, _NCH, chunk, 0)
    plsc.subcore_barrier()
    pltpu.sync_copy(acc.at[pl.ds(s * _RPT, _RPT)],
                    out_hbm.at[c, pl.ds(s * _RPT, _RPT)])


def _proj_body(x_ref, w_ref, o_ref):
    o_ref[...] = jnp.dot(x_ref[...], w_ref[...], preferred_element_type=jnp.float32)


def _proj(x, w):
    d = x.shape[1]
    return pl.pallas_call(
        _proj_body,
        grid=(_NB,),
        in_specs=[
            pl.BlockSpec((_BN, d), lambda i: (i, 0)),
            pl.BlockSpec((d, _H), lambda i: (0, 0)),
        ],
        out_specs=pl.BlockSpec((_BN, _H), lambda i: (i, 0)),
        out_shape=jax.ShapeDtypeStruct((_N, _H), jnp.float32),
    )(x, w)


def _mlp_body(has_next, p_ref, agg_ref, b1_ref, w2_ref, b2_ref, w1n_ref, h_ref,
              pn_ref=None):
    m = 2.0 * p_ref[...] + agg_ref[0] + agg_ref[1] + b1_ref[...]
    m = jnp.maximum(m, 0.0)
    h = jnp.dot(m, w2_ref[...], preferred_element_type=jnp.float32) + b2_ref[...]
    h = jnp.maximum(h, 0.0)
    h_ref[...] = h
    if has_next:
        pn_ref[...] = jnp.dot(h, w1n_ref[...], preferred_element_type=jnp.float32)


def _mlp(p, agg, b1, w2, b2, w1n, has_next):
    in_specs = [
        pl.BlockSpec((_BN, _H), lambda i: (i, 0)),
        pl.BlockSpec((_NSC, _BN, _H), lambda i: (0, i, 0)),
        pl.BlockSpec((1, _H), lambda i: (0, 0)),
        pl.BlockSpec((_H, _H), lambda i: (0, 0)),
        pl.BlockSpec((1, _H), lambda i: (0, 0)),
        pl.BlockSpec((_H, _H), lambda i: (0, 0)),
    ]
    if has_next:
        out_specs = [pl.BlockSpec((_BN, _H), lambda i: (i, 0)),
                     pl.BlockSpec((_BN, _H), lambda i: (i, 0))]
        out_shape = [jax.ShapeDtypeStruct((_N, _H), jnp.float32),
                     jax.ShapeDtypeStruct((_N, _H), jnp.float32)]
    else:
        out_specs = pl.BlockSpec((_BN, _H), lambda i: (i, 0))
        out_shape = jax.ShapeDtypeStruct((_N, _H), jnp.float32)
    return pl.pallas_call(
        functools.partial(_mlp_body, has_next),
        grid=(_NB,),
        in_specs=in_specs,
        out_specs=out_specs,
        out_shape=out_shape,
    )(p, agg, b1, w2, b2, w1n)


def _pool_head_body(h1_ref, h2_ref, h3_ref, bat_ref, fc1w_ref, fc1b_ref,
                    fc2w_ref, fc2b_ref, out_ref, s_acc, m_acc, c_acc):
    i = pl.program_id(0)

    @pl.when(i == 0)
    def _init():
        s_acc[...] = jnp.zeros_like(s_acc)
        m_acc[...] = jnp.full_like(m_acc, -jnp.inf)
        c_acc[...] = jnp.zeros_like(c_acc)

    X = jnp.concatenate([h1_ref[...], h2_ref[...], h3_ref[...]], axis=1)
    bb = bat_ref[...]
    onehot = (bb == lax.broadcasted_iota(jnp.int32, (_BN, _G), 1)
              ).astype(jnp.float32)
    s_acc[...] += lax.dot_general(onehot, X, (((0,), (0,)), ((), ())),
                                  preferred_element_type=jnp.float32)
    ones = jnp.ones((_BN, 8), jnp.float32)
    c_acc[...] += lax.dot_general(onehot, ones, (((0,), (0,)), ((), ())),
                                  preferred_element_type=jnp.float32)

    def g_body(g, carry):
        mask = bb == g
        mg = jnp.max(jnp.where(mask, X, -jnp.inf), axis=0, keepdims=True)
        m_acc[pl.ds(g, 1), :] = jnp.maximum(m_acc[pl.ds(g, 1), :], mg)
        return carry

    lax.fori_loop(0, _G, g_body, 0)

    @pl.when(i == _NB - 1)
    def _final():
        cnt = c_acc[:, 0:1]
        s = s_acc[...]
        mean = s / jnp.maximum(cnt, 1.0)
        mx = jnp.where(cnt > 0.0, m_acc[...], 0.0)
        w = fc1w_ref[...]
        z = (jnp.dot(mean, w[0:192], preferred_element_type=jnp.float32)
             + jnp.dot(mx, w[192:384], preferred_element_type=jnp.float32)
             + jnp.dot(s, w[384:576], preferred_element_type=jnp.float32)
             + fc1b_ref[...])
        z = jnp.maximum(z, 0.0)
        o = jnp.dot(z, fc2w_ref[...], preferred_element_type=jnp.float32) \
            + fc2b_ref[...]
        out_ref[...] = 1.0 / (1.0 + jnp.exp(-o))


def _pool_head(h1, h2, h3, bat3, fc1_w, fc1_b, fc2_w, fc2_b):
    return pl.pallas_call(
        _pool_head_body,
        grid=(_NB,),
        in_specs=[
            pl.BlockSpec((_BN, _H), lambda i: (i, 0)),
            pl.BlockSpec((_BN, _H), lambda i: (i, 0)),
            pl.BlockSpec((_BN, _H), lambda i: (i, 0)),
            pl.BlockSpec((_BN, 1), lambda i: (i, 0)),
            pl.BlockSpec((9 * _H, _H), lambda i: (0, 0)),
            pl.BlockSpec((1, _H), lambda i: (0, 0)),
            pl.BlockSpec((_H, _C), lambda i: (0, 0)),
            pl.BlockSpec((1, _C), lambda i: (0, 0)),
        ],
        out_specs=pl.BlockSpec((_G, _C), lambda i: (0, 0)),
        out_shape=jax.ShapeDtypeStruct((_G, _C), jnp.float32),
        scratch_shapes=[
            pltpu.VMEM((_G, 3 * _H), jnp.float32),
            pltpu.VMEM((_G, 3 * _H), jnp.float32),
            pltpu.VMEM((_G, 8), jnp.float32),
        ],
    )(h1, h2, h3, bat3, fc1_w, fc1_b, fc2_w, fc2_b)


def kernel(x, edge_index, batch,
           c0_w1, c0_b1, c0_w2, c0_b2,
           c1_w1, c1_b1, c1_w2, c1_b2,
           c2_w1, c2_b1, c2_w2, c2_b2,
           fc1_w, fc1_b, fc2_w, fc2_b):
    src = edge_index[0].astype(jnp.int32).reshape(_NW, _NCH, _K)
    dst = edge_index[1].astype(jnp.int32).reshape(_NW, _NCH, _K)
    zeros = jnp.zeros((_NP, _H), jnp.float32)
    bat3 = batch.astype(jnp.int32).reshape(_N, 1)

    params = [(c0_b1, c0_w2, c0_b2), (c1_b1, c1_w2, c1_b2), (c2_b1, c2_w2, c2_b2)]
    next_w1 = [c1_w1, c2_w1, None]

    p = _proj(x, c0_w1)
    hs = []
    for l in range(3):
        agg = _sc_agg(p, src, dst, zeros)
        b1, w2, b2 = params[l]
        has_next = next_w1[l] is not None
        w1n = next_w1[l] if has_next else w2
        res = _mlp(p, agg, b1.reshape(1, _H), w2, b2.reshape(1, _H), w1n,
                   has_next)
        if has_next:
            h, p = res
        else:
            h = res
        hs.append(h)

    return _pool_head(hs[0], hs[1], hs[2], bat3, fc1_w,
                      fc1_b.reshape(1, _H), fc2_w, fc2_b.reshape(1, _C))


# speedup vs baseline: 10.1556x; 1.4779x over previous
import functools

import jax
import jax.numpy as jnp
from jax import lax
from jax.experimental import pallas as pl
from jax.experimental.pallas import tpu as pltpu
from jax.experimental.pallas import tpu_sc as plsc

_N = 10000
_E = 320000
_D = 128
_H = 64
_G = 64
_C = 10

_NSC = 2
_NTILE = 16
_NW = _NSC * _NTILE
_EPT = _E // _NW
_K = 125
_NCH = _EPT // _K
_GSZ = 4
_NGRP = _NCH // _GSZ
_NP = 10240
_RPT = _NP // _NTILE

_NB = 10
_BN = _N // _NB


@functools.partial(
    pl.kernel,
    out_type=jax.ShapeDtypeStruct((_NSC, _NP, _H), jnp.float32),
    mesh=plsc.VectorSubcoreMesh(core_axis_name="c", subcore_axis_name="s"),
    scratch_types=[
        pltpu.VMEM((_NCH, _K), jnp.int32),
        pltpu.VMEM((_NCH, _K), jnp.int32),
        pltpu.VMEM((2, _GSZ, _K, _H), jnp.float32),
        pltpu.VMEM_SHARED((_NP, _H), jnp.float32),
        pltpu.SemaphoreType.DMA,
        pltpu.SemaphoreType.DMA,
        pltpu.SemaphoreType.DMA,
        pltpu.SemaphoreType.DMA,
    ],
    compiler_params=pltpu.CompilerParams(use_tc_tiling_on_sc=False),
)
def _sc_agg(p_hbm, src_hbm, dst_hbm, zero_hbm, out_hbm, srcv, dstv, rows, acc,
            sga, sgb, ssa, ssb):
    c = lax.axis_index("c")
    s = lax.axis_index("s")
    wid = c * _NTILE + s
    pltpu.sync_copy(zero_hbm.at[pl.ds(s * _RPT, _RPT)], acc.at[pl.ds(s * _RPT, _RPT)])
    pltpu.sync_copy(src_hbm.at[wid], srcv)
    pltpu.sync_copy(dst_hbm.at[wid], dstv)
    plsc.subcore_barrier()

    def fire(bank, g, sem):
        for t in range(_GSZ):
            pltpu.async_copy(p_hbm.at[srcv.at[g * _GSZ + t]],
                             rows.at[bank, t], sem)

    def drain(bank, g, semg, sems):
        for t in range(_GSZ):
            pltpu.make_async_copy(p_hbm.at[srcv.at[g * _GSZ + t]],
                                  rows.at[bank, t], semg).wait()
        for t in range(_GSZ):
            pltpu.async_copy(rows.at[bank, t],
                             acc.at[dstv.at[g * _GSZ + t]], sems, add=True)
        for t in range(_GSZ):
            pltpu.make_async_copy(rows.at[bank, t],
                                  acc.at[dstv.at[g * _GSZ + t]], sems).wait()

    fire(0, 0, sga)

    def body(i, carry):
        fire(1, 2 * i + 1, sgb)
        drain(0, 2 * i, sga, ssa)

        @pl.when(i < _NGRP // 2 - 1)
        def _next():
            fire(0, 2 * i + 2, sga)

        drain(1, 2 * i + 1, sgb, ssb)
        return carry

    lax.fori_loop(0, _NGRP // 2, body, 0)
    plsc.subcore_barrier()
    pltpu.sync_copy(acc.at[pl.ds(s * _RPT, _RPT)],
                    out_hbm.at[c, pl.ds(s * _RPT, _RPT)])


def _proj_body(x_ref, w_ref, o_ref):
    o_ref[...] = jnp.dot(x_ref[...], w_ref[...], preferred_element_type=jnp.float32)


def _proj(x, w):
    d = x.shape[1]
    return pl.pallas_call(
        _proj_body,
        grid=(_NB,),
        in_specs=[
            pl.BlockSpec((_BN, d), lambda i: (i, 0)),
            pl.BlockSpec((d, _H), lambda i: (0, 0)),
        ],
        out_specs=pl.BlockSpec((_BN, _H), lambda i: (i, 0)),
        out_shape=jax.ShapeDtypeStruct((_N, _H), jnp.float32),
    )(x, w)


def _mlp_body(has_next, p_ref, agg_ref, b1_ref, w2_ref, b2_ref, w1n_ref, h_ref,
              pn_ref=None):
    m = 2.0 * p_ref[...] + agg_ref[0] + agg_ref[1] + b1_ref[...]
    m = jnp.maximum(m, 0.0)
    h = jnp.dot(m, w2_ref[...], preferred_element_type=jnp.float32) + b2_ref[...]
    h = jnp.maximum(h, 0.0)
    h_ref[...] = h
    if has_next:
        pn_ref[...] = jnp.dot(h, w1n_ref[...], preferred_element_type=jnp.float32)


def _mlp(p, agg, b1, w2, b2, w1n, has_next):
    in_specs = [
        pl.BlockSpec((_BN, _H), lambda i: (i, 0)),
        pl.BlockSpec((_NSC, _BN, _H), lambda i: (0, i, 0)),
        pl.BlockSpec((1, _H), lambda i: (0, 0)),
        pl.BlockSpec((_H, _H), lambda i: (0, 0)),
        pl.BlockSpec((1, _H), lambda i: (0, 0)),
        pl.BlockSpec((_H, _H), lambda i: (0, 0)),
    ]
    if has_next:
        out_specs = [pl.BlockSpec((_BN, _H), lambda i: (i, 0)),
                     pl.BlockSpec((_BN, _H), lambda i: (i, 0))]
        out_shape = [jax.ShapeDtypeStruct((_N, _H), jnp.float32),
                     jax.ShapeDtypeStruct((_N, _H), jnp.float32)]
    else:
        out_specs = pl.BlockSpec((_BN, _H), lambda i: (i, 0))
        out_shape = jax.ShapeDtypeStruct((_N, _H), jnp.float32)
    return pl.pallas_call(
        functools.partial(_mlp_body, has_next),
        grid=(_NB,),
        in_specs=in_specs,
        out_specs=out_specs,
        out_shape=out_shape,
    )(p, agg, b1, w2, b2, w1n)


def _pool_head_body(h1_ref, h2_ref, h3_ref, bat_ref, fc1w_ref, fc1b_ref,
                    fc2w_ref, fc2b_ref, out_ref, s_acc, m_acc, c_acc):
    i = pl.program_id(0)

    @pl.when(i == 0)
    def _init():
        s_acc[...] = jnp.zeros_like(s_acc)
        m_acc[...] = jnp.full_like(m_acc, -jnp.inf)
        c_acc[...] = jnp.zeros_like(c_acc)

    X = jnp.concatenate([h1_ref[...], h2_ref[...], h3_ref[...]], axis=1)
    bb = bat_ref[...]
    onehot = (bb == lax.broadcasted_iota(jnp.int32, (_BN, _G), 1)
              ).astype(jnp.float32)
    s_acc[...] += lax.dot_general(onehot, X, (((0,), (0,)), ((), ())),
                                  preferred_element_type=jnp.float32)
    ones = jnp.ones((_BN, 8), jnp.float32)
    c_acc[...] += lax.dot_general(onehot, ones, (((0,), (0,)), ((), ())),
                                  preferred_element_type=jnp.float32)

    def g_body(g, carry):
        mask = bb == g
        mg = jnp.max(jnp.where(mask, X, -jnp.inf), axis=0, keepdims=True)
        m_acc[pl.ds(g, 1), :] = jnp.maximum(m_acc[pl.ds(g, 1), :], mg)
        return carry

    lax.fori_loop(0, _G, g_body, 0)

    @pl.when(i == _NB - 1)
    def _final():
        cnt = c_acc[:, 0:1]
        s = s_acc[...]
        mean = s / jnp.maximum(cnt, 1.0)
        mx = jnp.where(cnt > 0.0, m_acc[...], 0.0)
        w = fc1w_ref[...]
        z = (jnp.dot(mean, w[0:192], preferred_element_type=jnp.float32)
             + jnp.dot(mx, w[192:384], preferred_element_type=jnp.float32)
             + jnp.dot(s, w[384:576], preferred_element_type=jnp.float32)
             + fc1b_ref[...])
        z = jnp.maximum(z, 0.0)
        o = jnp.dot(z, fc2w_ref[...], preferred_element_type=jnp.float32) \
            + fc2b_ref[...]
        out_ref[...] = 1.0 / (1.0 + jnp.exp(-o))


def _pool_head(h1, h2, h3, bat3, fc1_w, fc1_b, fc2_w, fc2_b):
    return pl.pallas_call(
        _pool_head_body,
        grid=(_NB,),
        in_specs=[
            pl.BlockSpec((_BN, _H), lambda i: (i, 0)),
            pl.BlockSpec((_BN, _H), lambda i: (i, 0)),
            pl.BlockSpec((_BN, _H), lambda i: (i, 0)),
            pl.BlockSpec((_BN, 1), lambda i: (i, 0)),
            pl.BlockSpec((9 * _H, _H), lambda i: (0, 0)),
            pl.BlockSpec((1, _H), lambda i: (0, 0)),
            pl.BlockSpec((_H, _C), lambda i: (0, 0)),
            pl.BlockSpec((1, _C), lambda i: (0, 0)),
        ],
        out_specs=pl.BlockSpec((_G, _C), lambda i: (0, 0)),
        out_shape=jax.ShapeDtypeStruct((_G, _C), jnp.float32),
        scratch_shapes=[
            pltpu.VMEM((_G, 3 * _H), jnp.float32),
            pltpu.VMEM((_G, 3 * _H), jnp.float32),
            pltpu.VMEM((_G, 8), jnp.float32),
        ],
    )(h1, h2, h3, bat3, fc1_w, fc1_b, fc2_w, fc2_b)


def kernel(x, edge_index, batch,
           c0_w1, c0_b1, c0_w2, c0_b2,
           c1_w1, c1_b1, c1_w2, c1_b2,
           c2_w1, c2_b1, c2_w2, c2_b2,
           fc1_w, fc1_b, fc2_w, fc2_b):
    src = edge_index[0].astype(jnp.int32).reshape(_NW, _NCH, _K)
    dst = edge_index[1].astype(jnp.int32).reshape(_NW, _NCH, _K)
    zeros = jnp.zeros((_NP, _H), jnp.float32)
    bat3 = batch.astype(jnp.int32).reshape(_N, 1)

    params = [(c0_b1, c0_w2, c0_b2), (c1_b1, c1_w2, c1_b2), (c2_b1, c2_w2, c2_b2)]
    next_w1 = [c1_w1, c2_w1, None]

    p = _proj(x, c0_w1)
    hs = []
    for l in range(3):
        agg = _sc_agg(p, src, dst, zeros)
        b1, w2, b2 = params[l]
        has_next = next_w1[l] is not None
        w1n = next_w1[l] if has_next else w2
        res = _mlp(p, agg, b1.reshape(1, _H), w2, b2.reshape(1, _H), w1n,
                   has_next)
        if has_next:
            h, p = res
        else:
            h = res
        hs.append(h)

    return _pool_head(hs[0], hs[1], hs[2], bat3, fc1_w,
                      fc1_b.reshape(1, _H), fc2_w, fc2_b.reshape(1, _C))
